# Initial kernel scaffold; baseline (speedup 1.0000x reference)
import jax
import jax.numpy as jnp
from jax.experimental import pallas as pl


def kernel(center_ids, context_ids, W_in, W_out):
    raise NotImplementedError("write your pallas kernel here")


# speedup vs baseline: 7.6326x; 7.6326x over previous
import functools

import jax
import jax.numpy as jnp
from jax import lax
from jax.experimental import pallas as pl
from jax.experimental.pallas import tpu as pltpu
from jax.experimental.pallas import tpu_sc as plsc

DIM = 128
NCTX = 20
NPAD = 32
NW = 32
CH = 16
LANES = 16
DCH = DIM // LANES


def _body(cid_hbm, ctx_hbm, win_hbm, wout_hbm, out_hbm,
          cid_v, ctx_v, crows_v, xrows_v, logits_v, sem):
    b_per_w = cid_v.shape[0]
    n_chunks = b_per_w // CH
    wid = lax.axis_index("s") * 2 + lax.axis_index("c")
    base = wid * b_per_w

    pltpu.sync_copy(cid_hbm.at[pl.ds(base, b_per_w)], cid_v)
    pltpu.sync_copy(ctx_hbm.at[pl.ds(base * NCTX, b_per_w * NCTX)], ctx_v)

    lane = lax.iota(jnp.int32, LANES)
    zeros = jnp.zeros((LANES,), jnp.float32)

    def chunk_body(c, carry):
        coff = c * CH * NCTX
        cp0 = pltpu.async_copy(
            win_hbm.at[cid_v.at[pl.ds(c * CH, CH)]], crows_v, sem)
        cp1 = pltpu.async_copy(
            wout_hbm.at[ctx_v.at[pl.ds(coff, 128)]],
            xrows_v.at[pl.ds(0, 128)], sem)
        cp2 = pltpu.async_copy(
            wout_hbm.at[ctx_v.at[pl.ds(coff + 128, 128)]],
            xrows_v.at[pl.ds(128, 128)], sem)
        cp3 = pltpu.async_copy(
            wout_hbm.at[ctx_v.at[pl.ds(coff + 256, 64)]],
            xrows_v.at[pl.ds(256, 64)], sem)
        cp0.wait()
        cp1.wait()
        cp2.wait()
        cp3.wait()

        def elem_body(b, carry2):
            cvecs = [crows_v[b, pl.ds(LANES * k, LANES)] for k in range(DCH)]
            outs = [zeros, zeros]
            for n in range(NCTX):
                p = b * NCTX + n
                acc = cvecs[0] * xrows_v[p, pl.ds(0, LANES)]
                for k in range(1, DCH):
                    acc = acc + cvecs[k] * xrows_v[p, pl.ds(LANES * k, LANES)]
                s = jnp.sum(acc)
                g, j = divmod(n, LANES)
                outs[g] = jnp.where(lane == j, s, outs[g])
            logits_v[pl.ds(b * NPAD, LANES)] = outs[0]
            logits_v[pl.ds(b * NPAD + LANES, LANES)] = outs[1]
            return carry2

        lax.fori_loop(0, CH, elem_body, 0)
        pltpu.sync_copy(
            logits_v, out_hbm.at[pl.ds((base + c * CH) * NPAD, CH * NPAD)])
        return carry

    lax.fori_loop(0, n_chunks, chunk_body, 0)


def kernel(center_ids, context_ids, W_in, W_out):
    B = center_ids.shape[0]
    b_per_w = B // NW
    ctx_flat = context_ids.reshape(-1)

    run = functools.partial(
        pl.kernel,
        out_type=jax.ShapeDtypeStruct((B * NPAD,), jnp.float32),
        mesh=plsc.VectorSubcoreMesh(core_axis_name="c", subcore_axis_name="s"),
        compiler_params=pltpu.CompilerParams(needs_layout_passes=False),
        scratch_types=[
            pltpu.VMEM((b_per_w,), jnp.int32),
            pltpu.VMEM((b_per_w * NCTX,), jnp.int32),
            pltpu.VMEM((CH, DIM), jnp.float32),
            pltpu.VMEM((CH * NCTX, DIM), jnp.float32),
            pltpu.VMEM((CH * NPAD,), jnp.float32),
            pltpu.SemaphoreType.DMA,
        ],
    )(_body)
    out = run(center_ids, ctx_flat, W_in, W_out)
    return out.reshape(B, NPAD)[:, :NCTX]


# speedup vs baseline: 11.5853x; 1.5179x over previous
import functools

import jax
import jax.numpy as jnp
from jax import lax
from jax.experimental import pallas as pl
from jax.experimental.pallas import tpu as pltpu
from jax.experimental.pallas import tpu_sc as plsc

DIM = 128
NCTX = 20
NPAD = 32
NW = 32
CH = 16
LANES = 16
DCH = DIM // LANES


def _body(cid_hbm, ctx_hbm, win_hbm, wout_hbm, out_hbm,
          cid_v, ctx_v, cr0, cr1, xr0, xr1, logits_v, sem0, sem1):
    b_per_w = cid_v.shape[0]
    n_chunks = b_per_w // CH
    n_rounds = n_chunks // 2
    wid = lax.axis_index("s") * 2 + lax.axis_index("c")
    base = wid * b_per_w

    pltpu.sync_copy(cid_hbm.at[pl.ds(base, b_per_w)], cid_v)
    pltpu.sync_copy(ctx_hbm.at[pl.ds(base * NCTX, b_per_w * NCTX)], ctx_v)

    lane = lax.iota(jnp.int32, LANES)
    zeros = jnp.zeros((LANES,), jnp.float32)

    def fire(c, crb, xrb, sem):
        coff = c * CH * NCTX
        pltpu.async_copy(win_hbm.at[cid_v.at[pl.ds(c * CH, CH)]], crb, sem)
        pltpu.async_copy(wout_hbm.at[ctx_v.at[pl.ds(coff, 128)]],
                         xrb.at[pl.ds(0, 128)], sem)
        pltpu.async_copy(wout_hbm.at[ctx_v.at[pl.ds(coff + 128, 128)]],
                         xrb.at[pl.ds(128, 128)], sem)
        pltpu.async_copy(wout_hbm.at[ctx_v.at[pl.ds(coff + 256, 64)]],
                         xrb.at[pl.ds(256, 64)], sem)

    def drain(sem, crb, xrb):
        pltpu.make_async_copy(win_hbm.at[pl.ds(0, CH)], crb, sem).wait()
        pltpu.make_async_copy(wout_hbm.at[pl.ds(0, CH * NCTX)], xrb, sem).wait()

    def compute(c, crb, xrb):
        def elem_body(b, carry2):
            cvecs = [crb[b, pl.ds(LANES * k, LANES)] for k in range(DCH)]
            outs = [zeros, zeros]
            for n in range(NCTX):
                p = b * NCTX + n
                acc = cvecs[0] * xrb[p, pl.ds(0, LANES)]
                for k in range(1, DCH):
                    acc = acc + cvecs[k] * xrb[p, pl.ds(LANES * k, LANES)]
                s = jnp.sum(acc)
                g, j = divmod(n, LANES)
                outs[g] = jnp.where(lane == j, s, outs[g])
            logits_v[pl.ds(b * NPAD, LANES)] = outs[0]
            logits_v[pl.ds(b * NPAD + LANES, LANES)] = outs[1]
            return carry2

        lax.fori_loop(0, CH, elem_body, 0)
        pltpu.sync_copy(
            logits_v, out_hbm.at[pl.ds((base + c * CH) * NPAD, CH * NPAD)])

    fire(0, cr0, xr0, sem0)

    def round_body(r, carry):
        c0 = 2 * r
        fire(c0 + 1, cr1, xr1, sem1)
        drain(sem0, cr0, xr0)
        compute(c0, cr0, xr0)

        @pl.when(r < n_rounds - 1)
        def _():
            fire(c0 + 2, cr0, xr0, sem0)

        drain(sem1, cr1, xr1)
        compute(c0 + 1, cr1, xr1)
        return carry

    lax.fori_loop(0, n_rounds, round_body, 0)


def kernel(center_ids, context_ids, W_in, W_out):
    B = center_ids.shape[0]
    b_per_w = B // NW
    ctx_flat = context_ids.reshape(-1)

    run = functools.partial(
        pl.kernel,
        out_type=jax.ShapeDtypeStruct((B * NPAD,), jnp.float32),
        mesh=plsc.VectorSubcoreMesh(core_axis_name="c", subcore_axis_name="s"),
        compiler_params=pltpu.CompilerParams(needs_layout_passes=False),
        scratch_types=[
            pltpu.VMEM((b_per_w,), jnp.int32),
            pltpu.VMEM((b_per_w * NCTX,), jnp.int32),
            pltpu.VMEM((CH, DIM), jnp.float32),
            pltpu.VMEM((CH, DIM), jnp.float32),
            pltpu.VMEM((CH * NCTX, DIM), jnp.float32),
            pltpu.VMEM((CH * NCTX, DIM), jnp.float32),
            pltpu.VMEM((CH * NPAD,), jnp.float32),
            pltpu.SemaphoreType.DMA,
            pltpu.SemaphoreType.DMA,
        ],
    )(_body)
    out = run(center_ids, ctx_flat, W_in, W_out)
    return out.reshape(B, NPAD)[:, :NCTX]


# speedup vs baseline: 12.5384x; 1.0823x over previous
import functools

import jax
import jax.numpy as jnp
import numpy as np
from jax import lax
from jax.experimental import pallas as pl
from jax.experimental.pallas import tpu as pltpu
from jax.experimental.pallas import tpu_sc as plsc

DIM = 128
NCTX = 20
ROWPAD = 128
NW = 32
CH = 16
LANES = 16
DCH = DIM // LANES
NPAIR = CH * NCTX
NGRP = NPAIR // LANES

def _body(cid_hbm, ctxp_hbm, win_hbm, wout_hbm, out_hbm,
          cid_v, idp0, idp1, idx0, idx1, cr0, cr1, xr0, xr1, lg0, lg1,
          sid0, sid1, sr0, sr1, so0, so1):
    b_per_w = cid_v.shape[0]
    n_chunks = b_per_w // CH
    n_rounds = n_chunks // 2
    wid = lax.axis_index("s") * 2 + lax.axis_index("c")
    base = wid * b_per_w

    pltpu.sync_copy(cid_hbm.at[pl.ds(base, b_per_w)], cid_v)

    lane = lax.iota(jnp.int32, LANES)
    zeros = jnp.zeros((LANES,), jnp.float32)
    offs = []
    for g in range(NGRP):
        p = lane + (LANES * g)
        offs.append((p // NCTX) * ROWPAD + p % NCTX)

    def fire_ids(c, idp, sid):
        pltpu.async_copy(
            ctxp_hbm.at[pl.ds((base + c * CH) * ROWPAD, CH * ROWPAD)], idp, sid)

    def build_fire_rows(c, idp, sid, idx, crb, xrb, sr):
        pltpu.make_async_copy(
            ctxp_hbm.at[pl.ds(0, CH * ROWPAD)], idp, sid).wait()
        for g in range(NGRP):
            idx[pl.ds(LANES * g, LANES)] = plsc.load_gather(idp, [offs[g]])
        pltpu.async_copy(win_hbm.at[cid_v.at[pl.ds(c * CH, CH)]], crb, sr)
        pltpu.async_copy(wout_hbm.at[idx.at[pl.ds(0, 128)]],
                         xrb.at[pl.ds(0, 128)], sr)
        pltpu.async_copy(wout_hbm.at[idx.at[pl.ds(128, 128)]],
                         xrb.at[pl.ds(128, 128)], sr)
        pltpu.async_copy(wout_hbm.at[idx.at[pl.ds(256, 64)]],
                         xrb.at[pl.ds(256, 64)], sr)

    def compute(c, r, crb, xrb, lgb, sr, so):
        pltpu.make_async_copy(win_hbm.at[pl.ds(0, CH)], crb, sr).wait()
        pltpu.make_async_copy(wout_hbm.at[pl.ds(0, NPAIR)], xrb, sr).wait()

        @pl.when(r > 0)
        def _():
            pltpu.make_async_copy(
                lgb, out_hbm.at[pl.ds(0, CH * ROWPAD)], so).wait()

        def elem_body(b, carry2):
            cvecs = [crb[b, pl.ds(LANES * k, LANES)] for k in range(DCH)]
            outs = [zeros, zeros]
            for n in range(NCTX):
                p = b * NCTX + n
                acc = cvecs[0] * xrb[p, pl.ds(0, LANES)]
                for k in range(1, DCH):
                    acc = acc + cvecs[k] * xrb[p, pl.ds(LANES * k, LANES)]
                s = jnp.sum(acc)
                g, j = divmod(n, LANES)
                outs[g] = jnp.where(lane == j, s, outs[g])
            lgb[pl.ds(b * ROWPAD, LANES)] = outs[0]
            lgb[pl.ds(b * ROWPAD + LANES, LANES)] = outs[1]
            return carry2

        lax.fori_loop(0, CH, elem_body, 0)
        pltpu.async_copy(
            lgb, out_hbm.at[pl.ds((base + c * CH) * ROWPAD, CH * ROWPAD)], so)

    fire_ids(0, idp0, sid0)
    fire_ids(1, idp1, sid1)
    build_fire_rows(0, idp0, sid0, idx0, cr0, xr0, sr0)

    def round_body(r, carry):
        c0 = 2 * r
        build_fire_rows(c0 + 1, idp1, sid1, idx1, cr1, xr1, sr1)

        @pl.when(c0 + 2 < n_chunks)
        def _():
            fire_ids(c0 + 2, idp0, sid0)

        compute(c0, r, cr0, xr0, lg0, sr0, so0)

        @pl.when(c0 + 2 < n_chunks)
        def _():
            build_fire_rows(c0 + 2, idp0, sid0, idx0, cr0, xr0, sr0)

        @pl.when(c0 + 3 < n_chunks)
        def _():
            fire_ids(c0 + 3, idp1, sid1)

        compute(c0 + 1, r, cr1, xr1, lg1, sr1, so1)
        return carry

    lax.fori_loop(0, n_rounds, round_body, 0)

    pltpu.make_async_copy(lg0, out_hbm.at[pl.ds(0, CH * ROWPAD)], so0).wait()
    pltpu.make_async_copy(lg1, out_hbm.at[pl.ds(0, CH * ROWPAD)], so1).wait()


def kernel(center_ids, context_ids, W_in, W_out):
    B = center_ids.shape[0]
    b_per_w = B // NW
    ctx_pad = jnp.pad(context_ids, ((0, 0), (0, ROWPAD - NCTX)))
    ctx_flat = ctx_pad.reshape(-1)

    run = functools.partial(
        pl.kernel,
        out_type=jax.ShapeDtypeStruct((B * ROWPAD,), jnp.float32),
        mesh=plsc.VectorSubcoreMesh(core_axis_name="c", subcore_axis_name="s"),
        compiler_params=pltpu.CompilerParams(needs_layout_passes=False),
        scratch_types=[
            pltpu.VMEM((b_per_w,), jnp.int32),
            pltpu.VMEM((CH * ROWPAD,), jnp.int32),
            pltpu.VMEM((CH * ROWPAD,), jnp.int32),
            pltpu.VMEM((NPAIR,), jnp.int32),
            pltpu.VMEM((NPAIR,), jnp.int32),
            pltpu.VMEM((CH, DIM), jnp.float32),
            pltpu.VMEM((CH, DIM), jnp.float32),
            pltpu.VMEM((NPAIR, DIM), jnp.float32),
            pltpu.VMEM((NPAIR, DIM), jnp.float32),
            pltpu.VMEM((CH * ROWPAD,), jnp.float32),
            pltpu.VMEM((CH * ROWPAD,), jnp.float32),
            pltpu.SemaphoreType.DMA,
            pltpu.SemaphoreType.DMA,
            pltpu.SemaphoreType.DMA,
            pltpu.SemaphoreType.DMA,
            pltpu.SemaphoreType.DMA,
            pltpu.SemaphoreType.DMA,
        ],
    )(_body)
    out = run(center_ids, ctx_flat, W_in, W_out)
    return out.reshape(B, ROWPAD)[:, :NCTX]


# speedup vs baseline: 12.5479x; 1.0008x over previous
import functools

import jax
import jax.numpy as jnp
import numpy as np
from jax import lax
from jax.experimental import pallas as pl
from jax.experimental.pallas import tpu as pltpu
from jax.experimental.pallas import tpu_sc as plsc

DIM = 128
NCTX = 20
ROWPAD = 128
NW = 32
CH = 16
LANES = 16
DCH = DIM // LANES
NPAIR = CH * NCTX
NGRP = NPAIR // LANES

def _body(cid_hbm, ctxp_hbm, win_hbm, wout_hbm, out_hbm,
          cid_v, idp0, idp1, idx0, idx1, cr0, cr1, xr0, xr1, lg0, lg1,
          sid0, sid1, sr0, sr1, so0, so1):
    b_per_w = cid_v.shape[0]
    n_chunks = b_per_w // CH
    n_rounds = n_chunks // 2
    wid = lax.axis_index("s") * 2 + lax.axis_index("c")
    base = wid * b_per_w

    pltpu.sync_copy(cid_hbm.at[pl.ds(base, b_per_w)], cid_v)

    lane = lax.iota(jnp.int32, LANES)
    zeros = jnp.zeros((LANES,), jnp.float32)
    offs = []
    for g in range(NGRP):
        p = lane + (LANES * g)
        offs.append((p // NCTX) * ROWPAD + p % NCTX)

    def fire_ids(c, idp, sid):
        pltpu.async_copy(
            ctxp_hbm.at[pl.ds((base + c * CH) * ROWPAD, CH * ROWPAD)], idp, sid)

    def build_fire_rows(c, idp, sid, idx, crb, xrb, sr):
        pltpu.make_async_copy(
            ctxp_hbm.at[pl.ds(0, CH * ROWPAD)], idp, sid).wait()
        for g in range(NGRP):
            idx[pl.ds(LANES * g, LANES)] = plsc.load_gather(idp, [offs[g]])
        pltpu.async_copy(win_hbm.at[cid_v.at[pl.ds(c * CH, CH)]], crb, sr)
        pltpu.async_copy(wout_hbm.at[idx.at[pl.ds(0, 128)]],
                         xrb.at[pl.ds(0, 128)], sr)
        pltpu.async_copy(wout_hbm.at[idx.at[pl.ds(128, 128)]],
                         xrb.at[pl.ds(128, 128)], sr)
        pltpu.async_copy(wout_hbm.at[idx.at[pl.ds(256, 64)]],
                         xrb.at[pl.ds(256, 64)], sr)

    def compute(c, r, crb, xrb, lgb, sr, so):
        pltpu.make_async_copy(win_hbm.at[pl.ds(0, CH)], crb, sr).wait()
        pltpu.make_async_copy(wout_hbm.at[pl.ds(0, NPAIR)], xrb, sr).wait()

        @pl.when(r > 0)
        def _():
            pltpu.make_async_copy(
                lgb, out_hbm.at[pl.ds(0, CH * ROWPAD)], so).wait()

        def elem_body(b, carry2):
            cvecs = [crb[b, pl.ds(LANES * k, LANES)] for k in range(DCH)]
            outs = [zeros, zeros]
            for n in range(NCTX):
                p = b * NCTX + n
                acc = cvecs[0] * xrb[p, pl.ds(0, LANES)]
                for k in range(1, DCH):
                    acc = acc + cvecs[k] * xrb[p, pl.ds(LANES * k, LANES)]
                s = jnp.sum(acc)
                g, j = divmod(n, LANES)
                outs[g] = jnp.where(lane == j, s, outs[g])
            lgb[pl.ds(b * ROWPAD, LANES)] = outs[0]
            lgb[pl.ds(b * ROWPAD + LANES, LANES)] = outs[1]
            return carry2

        lax.fori_loop(0, CH, elem_body, 0)
        pltpu.async_copy(
            lgb, out_hbm.at[pl.ds((base + c * CH) * ROWPAD, CH * ROWPAD)], so)

    fire_ids(0, idp0, sid0)
    fire_ids(1, idp1, sid1)
    build_fire_rows(0, idp0, sid0, idx0, cr0, xr0, sr0)

    def round_body(r, carry):
        c0 = 2 * r
        build_fire_rows(c0 + 1, idp1, sid1, idx1, cr1, xr1, sr1)

        @pl.when(c0 + 2 < n_chunks)
        def _():
            fire_ids(c0 + 2, idp0, sid0)

        compute(c0, r, cr0, xr0, lg0, sr0, so0)

        @pl.when(c0 + 2 < n_chunks)
        def _():
            build_fire_rows(c0 + 2, idp0, sid0, idx0, cr0, xr0, sr0)

        @pl.when(c0 + 3 < n_chunks)
        def _():
            fire_ids(c0 + 3, idp1, sid1)

        compute(c0 + 1, r, cr1, xr1, lg1, sr1, so1)
        return carry

    lax.fori_loop(0, n_rounds, round_body, 0)

    pltpu.make_async_copy(lg0, out_hbm.at[pl.ds(0, CH * ROWPAD)], so0).wait()
    pltpu.make_async_copy(lg1, out_hbm.at[pl.ds(0, CH * ROWPAD)], so1).wait()


def kernel(center_ids, context_ids, W_in, W_out):
    B = center_ids.shape[0]
    b_per_w = B // NW
    ctx_pad = jnp.concatenate(
        [context_ids, jnp.zeros((B, ROWPAD - NCTX), jnp.int32)], axis=1)
    ctx_flat = ctx_pad.reshape(-1)

    run = functools.partial(
        pl.kernel,
        out_type=jax.ShapeDtypeStruct((B * ROWPAD,), jnp.float32),
        mesh=plsc.VectorSubcoreMesh(core_axis_name="c", subcore_axis_name="s"),
        compiler_params=pltpu.CompilerParams(needs_layout_passes=False),
        scratch_types=[
            pltpu.VMEM((b_per_w,), jnp.int32),
            pltpu.VMEM((CH * ROWPAD,), jnp.int32),
            pltpu.VMEM((CH * ROWPAD,), jnp.int32),
            pltpu.VMEM((NPAIR,), jnp.int32),
            pltpu.VMEM((NPAIR,), jnp.int32),
            pltpu.VMEM((CH, DIM), jnp.float32),
            pltpu.VMEM((CH, DIM), jnp.float32),
            pltpu.VMEM((NPAIR, DIM), jnp.float32),
            pltpu.VMEM((NPAIR, DIM), jnp.float32),
            pltpu.VMEM((CH * ROWPAD,), jnp.float32),
            pltpu.VMEM((CH * ROWPAD,), jnp.float32),
            pltpu.SemaphoreType.DMA,
            pltpu.SemaphoreType.DMA,
            pltpu.SemaphoreType.DMA,
            pltpu.SemaphoreType.DMA,
            pltpu.SemaphoreType.DMA,
            pltpu.SemaphoreType.DMA,
        ],
    )(_body)
    out = run(center_ids, ctx_flat, W_in, W_out)
    return out.reshape(B, ROWPAD)[:, :NCTX]
